# baseline (speedup 1.0000x reference)
import jax
import jax.numpy as jnp
from jax.experimental import pallas as pl

N = 10000
G = 16
H1 = 128
H2 = 64
HEADS = 8
NITER = 30


def _final_scale(h, mask):
    def body(h_ref, m_ref, o_ref):
        o_ref[...] = h_ref[...] * m_ref[...] + m_ref[...] * 1e-06

    return pl.pallas_call(
        body,
        out_shape=jax.ShapeDtypeStruct(h.shape, h.dtype),
    )(h, mask)


def kernel(x, edge_index, batch, tvol, p):
    row, col = edge_index[0], edge_index[1]
    xinit = x

    def get_mask(m):
        mm = jnp.abs(m).sum(-1) > 0
        neigh = jax.ops.segment_max(mm[row].astype(jnp.float32), col, num_segments=N)
        mm = jnp.logical_or(mm, neigh > 0)
        return mm.astype(jnp.float32)[:, None]

    def gin(h, W1, b1, W2, b2, g, bt):
        agg = h + jax.ops.segment_sum(h[row], col, num_segments=N)
        z = jax.nn.relu(agg @ W1 + b1)
        z = jax.nn.relu(z @ W2 + b2)
        return z * g + bt

    mask = get_mask(x)
    h = gin(x, p['c1_W1'], p['c1_b1'], p['c1_W2'], p['c1_b2'], p['c1_g'], p['c1_bt'])
    h = h * mask
    for i in range(3):
        h = h + gin(h, p['cv%d_W1' % i], p['cv%d_b1' % i], p['cv%d_W2' % i], p['cv%d_b2' % i], p['cv%d_g' % i], p['cv%d_bt' % i])
        mask = get_mask(mask)
        h = h * mask
        h = h * p['bn%d_g' % i] + p['bn%d_b' % i]
    hh = (h @ p['gat_W']).reshape(N, HEADS, H2)
    asrc = (hh * p['gat_asrc']).sum(-1)
    adst = (hh * p['gat_adst']).sum(-1)
    alpha = jax.nn.leaky_relu(asrc[row] + adst[col], 0.2)
    amax = jax.ops.segment_max(alpha, col, num_segments=N)
    amax = jnp.where(jnp.isfinite(amax), amax, 0.0)
    ea = jnp.exp(alpha - amax[col])
    den = jax.ops.segment_sum(ea, col, num_segments=N) + 1e-16
    coef = ea / den[col]
    h = jax.ops.segment_sum(hh[row] * coef[:, :, None], col, num_segments=N).reshape(N, HEADS * H2)
    mask = get_mask(mask)
    h = h * mask
    h = jax.nn.leaky_relu(h @ p['l1_W'] + p['l1_b'])
    h = h * mask
    h = h * p['bn2_g'] + p['bn2_b']
    h = jax.nn.leaky_relu(h @ p['l2_W'] + p['l2_b'])
    h = h * mask
    bmax = jax.ops.segment_max(h, batch, num_segments=N)
    bmin = -jax.ops.segment_max(-h, batch, num_segments=N)
    h = (h - bmin[batch]) / (bmax[batch] + 1e-06 - bmin[batch])
    h = _final_scale(h, mask)
    h = h + xinit
    deg = jnp.zeros((N,), jnp.float32).at[row].add(1.0)[:, None]
    totalvol = jax.ops.segment_sum(jax.lax.stop_gradient(deg) * jnp.ones_like(h), batch, num_segments=G) + 1e-06
    target = tvol * totalvol[:, 0]
    a = jnp.ones((G, 1), jnp.float32)
    for _ in range(NITER):
        keep = (a[batch] * h < 1).astype(jnp.float32)
        x_k = h * keep * mask
        d_k = deg * keep * mask
        d_nk = deg * (1 - keep) * mask
        diff = target[:, None] - jax.ops.segment_sum(d_nk, batch, num_segments=G)
        dot = jax.ops.segment_sum(x_k * d_k, batch, num_segments=G)
        a = diff / (dot + 1e-05)
    probs = jnp.clip(a[batch] * h * mask, 0.0, 1.0)
    expected_cut = jax.ops.segment_sum(probs * deg, batch, num_segments=G) - jax.ops.segment_sum(probs[row] * probs[col], batch[row], num_segments=G)
    return probs[:, 0], expected_cut


# speedup vs baseline: 1.0406x; 1.0406x over previous
import functools

import jax
import jax.numpy as jnp
from jax import lax
from jax.experimental import pallas as pl
from jax.experimental.pallas import tpu as pltpu
from jax.experimental.pallas import tpu_sc as plsc

N = 10000
E = 320000
G = 16
H1 = 128
H2 = 64
HEADS = 8
NITER = 30

_NC = 2
_NS = 16
_MESH = plsc.VectorSubcoreMesh(core_axis_name="c", subcore_axis_name="s")

_K = 200
_EPT = E // (_NC * _NS)
_NCH = _EPT // _K
_NPAD = 10240
_RPT = _NPAD // _NS


def _seg_sum_mat_partials(h, row, col, zeros):
    @functools.partial(
        pl.kernel,
        out_type=jax.ShapeDtypeStruct((_NC, _NPAD, H1), jnp.float32),
        mesh=_MESH,
        scratch_types=[
            pltpu.VMEM((_K,), jnp.int32),
            pltpu.VMEM((_K,), jnp.int32),
            pltpu.VMEM((_K, H1), jnp.float32),
            pltpu.VMEM_SHARED((_NPAD, H1), jnp.float32),
            pltpu.SemaphoreType.DMA,
        ],
    )
    def k(h_hbm, row_hbm, col_hbm, z_hbm, out_hbm, idx_r, idx_c, rows, acc, sem):
        c = lax.axis_index("c")
        s = lax.axis_index("s")
        r0 = s * _RPT
        pltpu.sync_copy(z_hbm.at[pl.ds(r0, _RPT)], acc.at[pl.ds(r0, _RPT)])
        plsc.subcore_barrier()
        base0 = c * (E // _NC) + s * _EPT

        def body(i, carry):
            base = base0 + i * _K
            pltpu.sync_copy(row_hbm.at[pl.ds(base, _K)], idx_r)
            pltpu.async_copy(h_hbm.at[idx_r], rows, sem).wait()
            pltpu.sync_copy(col_hbm.at[pl.ds(base, _K)], idx_c)
            pltpu.sync_copy(rows, acc.at[idx_c], add=True)
            return carry

        lax.fori_loop(0, _NCH, body, 0)
        plsc.subcore_barrier()
        pltpu.sync_copy(acc.at[pl.ds(r0, _RPT)], out_hbm.at[c, pl.ds(r0, _RPT)])

    return k(h, row, col, zeros)


def _final_scale(h, mask):
    def body(h_ref, m_ref, o_ref):
        o_ref[...] = h_ref[...] * m_ref[...] + m_ref[...] * 1e-06

    return pl.pallas_call(
        body,
        out_shape=jax.ShapeDtypeStruct(h.shape, h.dtype),
    )(h, mask)


def kernel(x, edge_index, batch, tvol, p):
    row, col = edge_index[0], edge_index[1]
    xinit = x

    zeros_mat = jnp.zeros((_NPAD, H1), jnp.float32)

    def get_mask(m):
        mm = jnp.abs(m).sum(-1) > 0
        neigh = jax.ops.segment_max(mm[row].astype(jnp.float32), col, num_segments=N)
        mm = jnp.logical_or(mm, neigh > 0)
        return mm.astype(jnp.float32)[:, None]

    def gin(h, W1, b1, W2, b2, g, bt):
        agg = h + jax.ops.segment_sum(h[row], col, num_segments=N)
        z = jax.nn.relu(agg @ W1 + b1)
        z = jax.nn.relu(z @ W2 + b2)
        return z * g + bt

    def gin_mat(h, W1, b1, W2, b2, g, bt):
        hpad = jnp.zeros((_NPAD, H1), jnp.float32).at[:N].set(h)
        partials = _seg_sum_mat_partials(hpad, row, col, zeros_mat)
        agg = h + partials[0, :N] + partials[1, :N]
        z = jax.nn.relu(agg @ W1 + b1)
        z = jax.nn.relu(z @ W2 + b2)
        return z * g + bt

    mask = get_mask(x)
    h = gin(x, p['c1_W1'], p['c1_b1'], p['c1_W2'], p['c1_b2'], p['c1_g'], p['c1_bt'])
    h = h * mask
    for i in range(3):
        h = h + gin_mat(h, p['cv%d_W1' % i], p['cv%d_b1' % i], p['cv%d_W2' % i], p['cv%d_b2' % i], p['cv%d_g' % i], p['cv%d_bt' % i])
        mask = get_mask(mask)
        h = h * mask
        h = h * p['bn%d_g' % i] + p['bn%d_b' % i]
    hh = (h @ p['gat_W']).reshape(N, HEADS, H2)
    asrc = (hh * p['gat_asrc']).sum(-1)
    adst = (hh * p['gat_adst']).sum(-1)
    alpha = jax.nn.leaky_relu(asrc[row] + adst[col], 0.2)
    amax = jax.ops.segment_max(alpha, col, num_segments=N)
    amax = jnp.where(jnp.isfinite(amax), amax, 0.0)
    ea = jnp.exp(alpha - amax[col])
    den = jax.ops.segment_sum(ea, col, num_segments=N) + 1e-16
    coef = ea / den[col]
    h = jax.ops.segment_sum(hh[row] * coef[:, :, None], col, num_segments=N).reshape(N, HEADS * H2)
    mask = get_mask(mask)
    h = h * mask
    h = jax.nn.leaky_relu(h @ p['l1_W'] + p['l1_b'])
    h = h * mask
    h = h * p['bn2_g'] + p['bn2_b']
    h = jax.nn.leaky_relu(h @ p['l2_W'] + p['l2_b'])
    h = h * mask
    bmax = jax.ops.segment_max(h, batch, num_segments=N)
    bmin = -jax.ops.segment_max(-h, batch, num_segments=N)
    h = (h - bmin[batch]) / (bmax[batch] + 1e-06 - bmin[batch])
    h = _final_scale(h, mask)
    h = h + xinit
    deg = jnp.zeros((N,), jnp.float32).at[row].add(1.0)[:, None]
    totalvol = jax.ops.segment_sum(jax.lax.stop_gradient(deg) * jnp.ones_like(h), batch, num_segments=G) + 1e-06
    target = tvol * totalvol[:, 0]
    a = jnp.ones((G, 1), jnp.float32)
    for _ in range(NITER):
        keep = (a[batch] * h < 1).astype(jnp.float32)
        x_k = h * keep * mask
        d_k = deg * keep * mask
        d_nk = deg * (1 - keep) * mask
        diff = target[:, None] - jax.ops.segment_sum(d_nk, batch, num_segments=G)
        dot = jax.ops.segment_sum(x_k * d_k, batch, num_segments=G)
        a = diff / (dot + 1e-05)
    probs = jnp.clip(a[batch] * h * mask, 0.0, 1.0)
    expected_cut = jax.ops.segment_sum(probs * deg, batch, num_segments=G) - jax.ops.segment_sum(probs[row] * probs[col], batch[row], num_segments=G)
    return probs[:, 0], expected_cut


# speedup vs baseline: 22.5430x; 21.6639x over previous
import functools

import jax
import jax.numpy as jnp
from jax import lax
from jax.experimental import pallas as pl
from jax.experimental.pallas import tpu as pltpu
from jax.experimental.pallas import tpu_sc as plsc

N = 10000
E = 320000
G = 16
H1 = 128
H2 = 64
HEADS = 8
NITER = 30

_NC = 2
_NS = 16
_MESH = plsc.VectorSubcoreMesh(core_axis_name="c", subcore_axis_name="s")

_NPAD = 10240
_RPT = _NPAD // _NS
_EPC = E // _NC
_EPT = E // (_NC * _NS)


def _wid(c, s):
    return s * _NC + c


_K1 = 2000
_NCH1 = _EPT // _K1


@functools.partial(
    pl.kernel,
    out_type=(
        jax.ShapeDtypeStruct((_NC, _NPAD), jnp.float32),
        jax.ShapeDtypeStruct((_NC, _NPAD), jnp.float32),
        jax.ShapeDtypeStruct((_NC, _NPAD), jnp.float32),
    ),
    mesh=_MESH,
    scratch_types=[
        pltpu.VMEM((_K1,), jnp.int32),
        pltpu.VMEM((_K1,), jnp.int32),
        pltpu.VMEM((_K1,), jnp.float32),
        pltpu.VMEM((_K1,), jnp.float32),
        pltpu.VMEM((_K1,), jnp.float32),
        pltpu.VMEM_SHARED((_NPAD,), jnp.float32),
        pltpu.VMEM_SHARED((_NPAD,), jnp.float32),
        pltpu.VMEM_SHARED((_NPAD,), jnp.float32),
        pltpu.SemaphoreType.DMA,
    ],
)
def _edge_init_kernel(x_hbm, row_hbm, col_hbm, zv_hbm,
                      outx_hbm, outm_hbm, outd_hbm,
                      idx_r, idx_c, xg, mm, ones, accx, accm, accd, sem):
    c = lax.axis_index("c")
    s = lax.axis_index("s")
    r0 = s * _RPT
    pltpu.sync_copy(zv_hbm.at[pl.ds(r0, _RPT)], accx.at[pl.ds(r0, _RPT)])
    pltpu.sync_copy(zv_hbm.at[pl.ds(r0, _RPT)], accm.at[pl.ds(r0, _RPT)])
    pltpu.sync_copy(zv_hbm.at[pl.ds(r0, _RPT)], accd.at[pl.ds(r0, _RPT)])
    onev = jnp.ones((16,), jnp.float32)
    for i in range(_K1 // 16):
        ones[pl.ds(i * 16, 16)] = onev
    plsc.subcore_barrier()
    base0 = c * _EPC + s * _EPT

    def body(i, carry):
        base = base0 + i * _K1
        pltpu.sync_copy(row_hbm.at[pl.ds(base, _K1)], idx_r)
        pltpu.async_copy(x_hbm.at[idx_r], xg, sem).wait()
        pltpu.sync_copy(col_hbm.at[pl.ds(base, _K1)], idx_c)

        def vb(j, carry2):
            v = xg[pl.ds(j * 16, 16)]
            mm[pl.ds(j * 16, 16)] = jnp.where(v != 0.0, 1.0, 0.0)
            return carry2

        lax.fori_loop(0, _K1 // 16, vb, 0)
        pltpu.sync_copy(xg, accx.at[idx_c], add=True)
        pltpu.sync_copy(mm, accm.at[idx_c], add=True)
        pltpu.sync_copy(ones, accd.at[idx_r], add=True)
        return carry

    lax.fori_loop(0, _NCH1, body, 0)
    plsc.subcore_barrier()
    pltpu.sync_copy(accx.at[pl.ds(r0, _RPT)], outx_hbm.at[c, pl.ds(r0, _RPT)])
    pltpu.sync_copy(accm.at[pl.ds(r0, _RPT)], outm_hbm.at[c, pl.ds(r0, _RPT)])
    pltpu.sync_copy(accd.at[pl.ds(r0, _RPT)], outd_hbm.at[c, pl.ds(r0, _RPT)])


_K2 = 200
_NCH2 = _EPT // _K2


@functools.partial(
    pl.kernel,
    out_type=(
        jax.ShapeDtypeStruct((_NC, _NPAD, H1), jnp.float32),
        jax.ShapeDtypeStruct((_NC, _NPAD), jnp.float32),
    ),
    mesh=_MESH,
    scratch_types=[
        pltpu.VMEM((_K2,), jnp.int32),
        pltpu.VMEM((_K2,), jnp.int32),
        pltpu.VMEM((_K2, H1), jnp.float32),
        pltpu.VMEM((_K2,), jnp.float32),
        pltpu.VMEM_SHARED((_NPAD, H1), jnp.float32),
        pltpu.VMEM_SHARED((_NPAD,), jnp.float32),
        pltpu.SemaphoreType.DMA,
        pltpu.SemaphoreType.DMA,
    ],
)
def _gin_mask_kernel(h_hbm, m_hbm, row_hbm, col_hbm, zm_hbm, zv_hbm,
                     outa_hbm, outm_hbm,
                     idx_r, idx_c, rows, mg, acc, accm, sem, sem2):
    c = lax.axis_index("c")
    s = lax.axis_index("s")
    r0 = s * _RPT
    pltpu.sync_copy(zm_hbm.at[pl.ds(r0, _RPT)], acc.at[pl.ds(r0, _RPT)])
    pltpu.sync_copy(zv_hbm.at[pl.ds(r0, _RPT)], accm.at[pl.ds(r0, _RPT)])
    plsc.subcore_barrier()
    base0 = c * _EPC + s * _EPT

    def body(i, carry):
        base = base0 + i * _K2
        pltpu.sync_copy(row_hbm.at[pl.ds(base, _K2)], idx_r)
        cp1 = pltpu.async_copy(h_hbm.at[idx_r], rows, sem)
        cp2 = pltpu.async_copy(m_hbm.at[idx_r], mg, sem2)
        pltpu.sync_copy(col_hbm.at[pl.ds(base, _K2)], idx_c)
        cp1.wait()
        cp2.wait()
        pltpu.sync_copy(rows, acc.at[idx_c], add=True)
        pltpu.sync_copy(mg, accm.at[idx_c], add=True)
        return carry

    lax.fori_loop(0, _NCH2, body, 0)
    plsc.subcore_barrier()
    pltpu.sync_copy(acc.at[pl.ds(r0, _RPT)], outa_hbm.at[c, pl.ds(r0, _RPT)])
    pltpu.sync_copy(accm.at[pl.ds(r0, _RPT)], outm_hbm.at[c, pl.ds(r0, _RPT)])


_K3 = 1000
_NCH3 = _EPT // _K3


@functools.partial(
    pl.kernel,
    out_type=(
        jax.ShapeDtypeStruct((E, 16), jnp.float32),
        jax.ShapeDtypeStruct((_NC, _NPAD, 16), jnp.float32),
        jax.ShapeDtypeStruct((_NC, _NPAD), jnp.float32),
    ),
    mesh=_MESH,
    compiler_params=pltpu.CompilerParams(use_tc_tiling_on_sc=False),
    scratch_types=[
        pltpu.VMEM((_K3,), jnp.int32),
        pltpu.VMEM((_K3,), jnp.int32),
        pltpu.VMEM((_K3, 16), jnp.float32),
        pltpu.VMEM((_K3, 16), jnp.float32),
        pltpu.VMEM((_K3,), jnp.float32),
        pltpu.VMEM_SHARED((_NPAD, 16), jnp.float32),
        pltpu.VMEM_SHARED((_NPAD,), jnp.float32),
        pltpu.SemaphoreType.DMA,
        pltpu.SemaphoreType.DMA,
        pltpu.SemaphoreType.DMA,
    ],
)
def _gat1_kernel(asrc_hbm, adst_hbm, m_hbm, row_hbm, col_hbm, z16_hbm, zv_hbm,
                 ea_hbm, den_hbm, outm_hbm,
                 idx_r, idx_c, ar, bc, mg, accd, accm, sem, sem2, sem3):
    c = lax.axis_index("c")
    s = lax.axis_index("s")
    r0 = s * _RPT
    pltpu.sync_copy(z16_hbm.at[pl.ds(r0, _RPT)], accd.at[pl.ds(r0, _RPT)])
    pltpu.sync_copy(zv_hbm.at[pl.ds(r0, _RPT)], accm.at[pl.ds(r0, _RPT)])
    plsc.subcore_barrier()
    base0 = c * _EPC + s * _EPT

    def body(i, carry):
        base = base0 + i * _K3
        pltpu.sync_copy(row_hbm.at[pl.ds(base, _K3)], idx_r)
        pltpu.sync_copy(col_hbm.at[pl.ds(base, _K3)], idx_c)
        cp1 = pltpu.async_copy(asrc_hbm.at[idx_r], ar, sem)
        cp2 = pltpu.async_copy(adst_hbm.at[idx_c], bc, sem2)
        cp3 = pltpu.async_copy(m_hbm.at[idx_r], mg, sem3)
        cp1.wait()
        cp2.wait()
        cp3.wait()

        def vb(e, carry2):
            z = ar[e, :] + bc[e, :]
            z = jnp.where(z > 0.0, z, z * 0.2)
            ar[e, :] = jnp.exp(z)
            return carry2

        lax.fori_loop(0, _K3, vb, 0)
        pltpu.sync_copy(ar, accd.at[idx_c], add=True)
        pltpu.sync_copy(mg, accm.at[idx_c], add=True)
        pltpu.sync_copy(ar, ea_hbm.at[pl.ds(base, _K3)])
        return carry

    lax.fori_loop(0, _NCH3, body, 0)
    plsc.subcore_barrier()
    pltpu.sync_copy(accd.at[pl.ds(r0, _RPT)], den_hbm.at[c, pl.ds(r0, _RPT)])
    pltpu.sync_copy(accm.at[pl.ds(r0, _RPT)], outm_hbm.at[c, pl.ds(r0, _RPT)])


_K4 = 200
_NCH4 = _EPT // _K4


@functools.partial(
    pl.kernel,
    out_type=jax.ShapeDtypeStruct((_NC, _NPAD, H1), jnp.float32),
    mesh=_MESH,
    scratch_types=[
        pltpu.VMEM((_K4,), jnp.int32),
        pltpu.VMEM((_K4,), jnp.int32),
        pltpu.VMEM((_K4, H1), jnp.float32),
        pltpu.VMEM((_K4 * 2,), jnp.float32),
        pltpu.VMEM_SHARED((_NPAD, H1), jnp.float32),
        pltpu.SemaphoreType.DMA,
    ],
)
def _gat2_kernel(hh_hbm, ea2_hbm, row_hbm, col_hbm, zm_hbm,
                 out_hbm, idx_r, idx_c, rows2, ea2, acc, sem):
    c = lax.axis_index("c")
    s = lax.axis_index("s")
    r0 = s * _RPT
    pltpu.sync_copy(zm_hbm.at[pl.ds(r0, _RPT)], acc.at[pl.ds(r0, _RPT)])
    plsc.subcore_barrier()
    base0 = c * _EPC + s * _EPT

    def body(i, carry):
        base = base0 + i * _K4
        pltpu.sync_copy(row_hbm.at[pl.ds(base, _K4)], idx_r)
        cp1 = pltpu.async_copy(hh_hbm.at[idx_r], rows2, sem)
        pltpu.sync_copy(ea2_hbm.at[pl.ds(base * 2, _K4 * 2)], ea2)
        pltpu.sync_copy(col_hbm.at[pl.ds(base, _K4)], idx_c)
        cp1.wait()

        def eb(e8, carry2):
            ev = ea2[pl.ds(e8 * 16, 16)]
            for el in range(8):
                e = e8 * 8 + el
                s0 = jnp.full((16,), 0.0, jnp.float32) + ev[2 * el]
                s1 = jnp.full((16,), 0.0, jnp.float32) + ev[2 * el + 1]
                for j in range(4):
                    rows2[e, pl.ds(j * 16, 16)] = rows2[e, pl.ds(j * 16, 16)] * s0
                for j in range(4, 8):
                    rows2[e, pl.ds(j * 16, 16)] = rows2[e, pl.ds(j * 16, 16)] * s1
            return carry2

        lax.fori_loop(0, _K4 // 8, eb, 0)
        pltpu.sync_copy(rows2, acc.at[idx_c], add=True)
        return carry

    lax.fori_loop(0, _NCH4, body, 0)
    plsc.subcore_barrier()
    pltpu.sync_copy(acc.at[pl.ds(r0, _RPT)], out_hbm.at[c, pl.ds(r0, _RPT)])


_K5 = 1000
_NCH5 = _EPT // _K5


@functools.partial(
    pl.kernel,
    out_type=jax.ShapeDtypeStruct((_NC, 16, 16), jnp.float32),
    mesh=_MESH,
    compiler_params=pltpu.CompilerParams(use_tc_tiling_on_sc=False),
    scratch_types=[
        pltpu.VMEM((_K5,), jnp.int32),
        pltpu.VMEM((_K5,), jnp.int32),
        pltpu.VMEM((_K5, 16), jnp.float32),
        pltpu.VMEM((_K5, 16), jnp.float32),
        pltpu.VMEM((_K5,), jnp.int32),
        pltpu.VMEM((16, 16), jnp.float32),
        pltpu.VMEM_SHARED((16, 16), jnp.float32),
        pltpu.SemaphoreType.DMA,
        pltpu.SemaphoreType.DMA,
        pltpu.SemaphoreType.DMA,
    ],
)
def _cut_kernel(p_hbm, b_hbm, row_hbm, col_hbm,
                out_hbm, idx_r, idx_c, pr, pc, bg, zv, acc, sem, sem2, sem3):
    c = lax.axis_index("c")
    s = lax.axis_index("s")
    zvec16 = jnp.zeros((16,), jnp.float32)
    for i in range(16):
        zv[i, :] = zvec16

    @pl.when(s == 0)
    def _():
        pltpu.sync_copy(zv, acc)

    plsc.subcore_barrier()
    base0 = c * _EPC + s * _EPT

    def body(i, carry):
        base = base0 + i * _K5
        pltpu.sync_copy(row_hbm.at[pl.ds(base, _K5)], idx_r)
        pltpu.sync_copy(col_hbm.at[pl.ds(base, _K5)], idx_c)
        cp1 = pltpu.async_copy(p_hbm.at[idx_r], pr, sem)
        cp2 = pltpu.async_copy(p_hbm.at[idx_c], pc, sem2)
        cp3 = pltpu.async_copy(b_hbm.at[idx_r], bg, sem3)
        cp1.wait()
        cp2.wait()
        cp3.wait()

        def vb(e, carry2):
            pr[e, :] = pr[e, :] * pc[e, :]
            return carry2

        lax.fori_loop(0, _K5, vb, 0)
        pltpu.sync_copy(pr, acc.at[bg], add=True)
        return carry

    lax.fori_loop(0, _NCH5, body, 0)
    plsc.subcore_barrier()

    @pl.when(s == 0)
    def _():
        pltpu.sync_copy(acc, out_hbm.at[c])


def _niter_body(h_ref, mask_ref, xinit_ref, deg_ref, oh_ref, oht_ref,
                tvol_ref, probs_ref, cut1_ref):
    hv = h_ref[...]
    mv = mask_ref[...]
    xv = xinit_ref[...]
    dv = deg_ref[...]
    oh = oh_ref[...]
    oht = oht_ref[...]
    neg = jnp.float32(-jnp.inf)
    bmaxn = jnp.zeros_like(hv)
    bminn = jnp.zeros_like(hv)
    for g in range(G):
        sel = oht[g:g + 1, :]
        mg = jnp.max(jnp.where(sel > 0.0, hv, neg))
        mg = jnp.where(jnp.isfinite(mg), mg, 0.0)
        ng = -jnp.max(jnp.where(sel > 0.0, -hv, neg))
        ng = jnp.where(jnp.isfinite(ng), ng, 0.0)
        bmaxn = bmaxn + mg * sel
        bminn = bminn + ng * sel
    hv = (hv - bminn) / (bmaxn + 1e-06 - bminn)
    hv = hv * mv + mv * 1e-06 + xv
    totalvol = jnp.dot(dv, oh, preferred_element_type=jnp.float32) + 1e-06
    target = tvol_ref[...] * totalvol

    def it(i, a):
        an = jnp.dot(a, oht, preferred_element_type=jnp.float32)
        keep = (an * hv < 1.0).astype(jnp.float32)
        km = keep * mv
        xk = hv * km
        dk = dv * km
        dnk = dv * (1.0 - keep) * mv
        diff = target - jnp.dot(dnk, oh, preferred_element_type=jnp.float32)
        dot = jnp.dot(xk * dk, oh, preferred_element_type=jnp.float32)
        return diff / (dot + 1e-05)

    a = lax.fori_loop(0, NITER, it, jnp.ones((1, G), jnp.float32))
    an = jnp.dot(a, oht, preferred_element_type=jnp.float32)
    probs = jnp.clip(an * hv * mv, 0.0, 1.0)
    probs_ref[...] = probs
    cut1_ref[...] = jnp.dot(probs * dv, oh, preferred_element_type=jnp.float32)


def _niter_tc(h2, maskf, xinitf, degf, onehot, onehotT, tvol2):
    return pl.pallas_call(
        _niter_body,
        out_shape=(
            jax.ShapeDtypeStruct((1, _NPAD), jnp.float32),
            jax.ShapeDtypeStruct((1, G), jnp.float32),
        ),
    )(h2, maskf, xinitf, degf, onehot, onehotT, tvol2)


def kernel(x, edge_index, batch, tvol, p):
    row, col = edge_index[0], edge_index[1]
    xinit = x

    zvec = jnp.zeros((_NPAD,), jnp.float32)
    zmat = jnp.zeros((_NPAD, H1), jnp.float32)
    z16 = jnp.zeros((_NPAD, 16), jnp.float32)

    def padv(v):
        return jnp.zeros((_NPAD,), jnp.float32).at[:N].set(v)

    def padm(m):
        return jnp.zeros((_NPAD, H1), jnp.float32).at[:N].set(m)

    xflat = padv(x[:, 0])
    xaggp, mmp, degp = _edge_init_kernel(xflat, row, col, zvec)
    mm0 = jnp.abs(x[:, 0]) > 0
    neigh = mmp[0, :N] + mmp[1, :N]
    mask = jnp.logical_or(mm0, neigh > 0).astype(jnp.float32)[:, None]
    deg = (degp[0, :N] + degp[1, :N])[:, None]

    def mlp(agg, W1, b1, W2, b2, g, bt):
        z = jax.nn.relu(agg @ W1 + b1)
        z = jax.nn.relu(z @ W2 + b2)
        return z * g + bt

    xagg = x + (xaggp[0, :N] + xaggp[1, :N])[:, None]
    h = mlp(xagg, p['c1_W1'], p['c1_b1'], p['c1_W2'], p['c1_b2'], p['c1_g'], p['c1_bt'])
    h = h * mask

    for i in range(3):
        aggp, mmp = _gin_mask_kernel(padm(h), padv(mask[:, 0]), row, col, zmat, zvec)
        agg = h + aggp[0, :N] + aggp[1, :N]
        gi = mlp(agg, p['cv%d_W1' % i], p['cv%d_b1' % i], p['cv%d_W2' % i],
                 p['cv%d_b2' % i], p['cv%d_g' % i], p['cv%d_bt' % i])
        h = h + gi
        neigh = mmp[0, :N] + mmp[1, :N]
        mask = jnp.logical_or(mask[:, 0] > 0, neigh > 0).astype(jnp.float32)[:, None]
        h = h * mask
        h = h * p['bn%d_g' % i] + p['bn%d_b' % i]

    hh = (h @ p['gat_W']).reshape(N, HEADS, H2)
    asrc = (hh * p['gat_asrc']).sum(-1)
    adst = (hh * p['gat_adst']).sum(-1)
    a16 = jnp.zeros((_NPAD, 16), jnp.float32).at[:N].set(jnp.tile(asrc, (1, 2)))
    b16 = jnp.zeros((_NPAD, 16), jnp.float32).at[:N].set(jnp.tile(adst, (1, 2)))
    eaf, denp, mmp = _gat1_kernel(a16, b16, padv(mask[:, 0]), row, col, z16, zvec)
    den = denp[0, :N, :HEADS] + denp[1, :N, :HEADS] + 1e-16
    neigh = mmp[0, :N] + mmp[1, :N]
    mask = jnp.logical_or(mask[:, 0] > 0, neigh > 0).astype(jnp.float32)[:, None]

    num = []
    for cnk in range(4):
        hh_c = jnp.zeros((_NPAD, H1), jnp.float32).at[:N].set(
            hh[:, 2 * cnk:2 * cnk + 2, :].reshape(N, H1))
        ea2 = eaf[:, 2 * cnk:2 * cnk + 2].reshape(E * 2)
        np_ = _gat2_kernel(hh_c, ea2, row, col, zmat)
        num.append(np_[0, :N] + np_[1, :N])
    numf = jnp.concatenate(num, axis=1).reshape(N, HEADS, H2)
    h = (numf / den[:, :, None]).reshape(N, HEADS * H2)
    h = h * mask
    h = jax.nn.leaky_relu(h @ p['l1_W'] + p['l1_b'])
    h = h * mask
    h = h * p['bn2_g'] + p['bn2_b']
    h = jax.nn.leaky_relu(h @ p['l2_W'] + p['l2_b'])
    h = h * mask

    onehot = (batch[:, None] == jnp.arange(G)[None, :]).astype(jnp.float32)
    onehot = jnp.zeros((_NPAD, G), jnp.float32).at[:N].set(onehot)
    onehotT = onehot.T
    probs2, cut1 = _niter_tc(
        padv(h[:, 0]).reshape(1, _NPAD),
        padv(mask[:, 0]).reshape(1, _NPAD),
        padv(xinit[:, 0]).reshape(1, _NPAD),
        padv(deg[:, 0]).reshape(1, _NPAD),
        onehot, onehotT, tvol.reshape(1, G))
    probs = probs2[0, :N]

    bpad = jnp.zeros((_NPAD,), jnp.int32).at[:N].set(batch)
    p16 = jnp.tile(probs2.reshape(_NPAD, 1), (1, 16))
    cutp = _cut_kernel(p16, bpad, row, col)
    cut2 = cutp[0, :, 0] + cutp[1, :, 0]
    expected_cut = cut1[0][:, None] - cut2[:, None]
    return probs, expected_cut


# speedup vs baseline: 28.4733x; 1.2631x over previous
import functools

import jax
import jax.numpy as jnp
from jax import lax
from jax.experimental import pallas as pl
from jax.experimental.pallas import tpu as pltpu
from jax.experimental.pallas import tpu_sc as plsc

N = 10000
E = 320000
G = 16
H1 = 128
H2 = 64
HEADS = 8
NITER = 30

_NC = 2
_NS = 16
_MESH = plsc.VectorSubcoreMesh(core_axis_name="c", subcore_axis_name="s")

_NPAD = 10240
_RPT = _NPAD // _NS
_EPC = E // _NC
_EPT = E // (_NC * _NS)


def _wid(c, s):
    return s * _NC + c


_K1 = 2000
_NCH1 = _EPT // _K1


@functools.partial(
    pl.kernel,
    out_type=(
        jax.ShapeDtypeStruct((_NC, _NPAD), jnp.float32),
        jax.ShapeDtypeStruct((_NC, _NPAD), jnp.float32),
        jax.ShapeDtypeStruct((_NC, _NPAD), jnp.float32),
    ),
    mesh=_MESH,
    scratch_types=[
        pltpu.VMEM((_K1,), jnp.int32),
        pltpu.VMEM((_K1,), jnp.int32),
        pltpu.VMEM((_K1,), jnp.float32),
        pltpu.VMEM((_K1,), jnp.float32),
        pltpu.VMEM((_K1,), jnp.float32),
        pltpu.VMEM_SHARED((_NPAD,), jnp.float32),
        pltpu.VMEM_SHARED((_NPAD,), jnp.float32),
        pltpu.VMEM_SHARED((_NPAD,), jnp.float32),
        pltpu.SemaphoreType.DMA,
    ],
)
def _edge_init_kernel(x_hbm, row_hbm, col_hbm, zv_hbm,
                      outx_hbm, outm_hbm, outd_hbm,
                      idx_r, idx_c, xg, mm, ones, accx, accm, accd, sem):
    c = lax.axis_index("c")
    s = lax.axis_index("s")
    r0 = s * _RPT
    pltpu.sync_copy(zv_hbm.at[pl.ds(r0, _RPT)], accx.at[pl.ds(r0, _RPT)])
    pltpu.sync_copy(zv_hbm.at[pl.ds(r0, _RPT)], accm.at[pl.ds(r0, _RPT)])
    pltpu.sync_copy(zv_hbm.at[pl.ds(r0, _RPT)], accd.at[pl.ds(r0, _RPT)])
    onev = jnp.ones((16,), jnp.float32)
    for i in range(_K1 // 16):
        ones[pl.ds(i * 16, 16)] = onev
    plsc.subcore_barrier()
    base0 = c * _EPC + s * _EPT

    def body(i, carry):
        base = base0 + i * _K1
        pltpu.sync_copy(row_hbm.at[pl.ds(base, _K1)], idx_r)
        pltpu.async_copy(x_hbm.at[idx_r], xg, sem).wait()
        pltpu.sync_copy(col_hbm.at[pl.ds(base, _K1)], idx_c)

        def vb(j, carry2):
            v = xg[pl.ds(j * 16, 16)]
            mm[pl.ds(j * 16, 16)] = jnp.where(v != 0.0, 1.0, 0.0)
            return carry2

        lax.fori_loop(0, _K1 // 16, vb, 0)
        pltpu.sync_copy(xg, accx.at[idx_c], add=True)
        pltpu.sync_copy(mm, accm.at[idx_c], add=True)
        pltpu.sync_copy(ones, accd.at[idx_r], add=True)
        return carry

    lax.fori_loop(0, _NCH1, body, 0)
    plsc.subcore_barrier()
    pltpu.sync_copy(accx.at[pl.ds(r0, _RPT)], outx_hbm.at[c, pl.ds(r0, _RPT)])
    pltpu.sync_copy(accm.at[pl.ds(r0, _RPT)], outm_hbm.at[c, pl.ds(r0, _RPT)])
    pltpu.sync_copy(accd.at[pl.ds(r0, _RPT)], outd_hbm.at[c, pl.ds(r0, _RPT)])


_K2 = 200
_NCH2 = _EPT // _K2


@functools.partial(
    pl.kernel,
    out_type=(
        jax.ShapeDtypeStruct((_NC, _NPAD, H1), jnp.float32),
        jax.ShapeDtypeStruct((_NC, _NPAD), jnp.float32),
    ),
    mesh=_MESH,
    scratch_types=[
        pltpu.VMEM((_K2,), jnp.int32),
        pltpu.VMEM((_K2,), jnp.int32),
        pltpu.VMEM((_K2, H1), jnp.float32),
        pltpu.VMEM((_K2,), jnp.float32),
        pltpu.VMEM_SHARED((_NPAD, H1), jnp.float32),
        pltpu.VMEM_SHARED((_NPAD,), jnp.float32),
        pltpu.SemaphoreType.DMA,
        pltpu.SemaphoreType.DMA,
    ],
)
def _gin_mask_kernel(h_hbm, m_hbm, row_hbm, col_hbm, zm_hbm, zv_hbm,
                     outa_hbm, outm_hbm,
                     idx_r, idx_c, rows, mg, acc, accm, sem, sem2):
    c = lax.axis_index("c")
    s = lax.axis_index("s")
    r0 = s * _RPT
    pltpu.sync_copy(zm_hbm.at[pl.ds(r0, _RPT)], acc.at[pl.ds(r0, _RPT)])
    pltpu.sync_copy(zv_hbm.at[pl.ds(r0, _RPT)], accm.at[pl.ds(r0, _RPT)])
    plsc.subcore_barrier()
    base0 = c * _EPC + s * _EPT

    def body(i, carry):
        base = base0 + i * _K2
        pltpu.sync_copy(row_hbm.at[pl.ds(base, _K2)], idx_r)
        cp1 = pltpu.async_copy(h_hbm.at[idx_r], rows, sem)
        cp2 = pltpu.async_copy(m_hbm.at[idx_r], mg, sem2)
        pltpu.sync_copy(col_hbm.at[pl.ds(base, _K2)], idx_c)
        cp1.wait()
        cp2.wait()
        pltpu.sync_copy(rows, acc.at[idx_c], add=True)
        pltpu.sync_copy(mg, accm.at[idx_c], add=True)
        return carry

    lax.fori_loop(0, _NCH2, body, 0)
    plsc.subcore_barrier()
    pltpu.sync_copy(acc.at[pl.ds(r0, _RPT)], outa_hbm.at[c, pl.ds(r0, _RPT)])
    pltpu.sync_copy(accm.at[pl.ds(r0, _RPT)], outm_hbm.at[c, pl.ds(r0, _RPT)])


_K3 = 1000
_NCH3 = _EPT // _K3


@functools.partial(
    pl.kernel,
    out_type=(
        jax.ShapeDtypeStruct((E, 16), jnp.float32),
        jax.ShapeDtypeStruct((_NC, _NPAD, 16), jnp.float32),
        jax.ShapeDtypeStruct((_NC, _NPAD), jnp.float32),
    ),
    mesh=_MESH,
    compiler_params=pltpu.CompilerParams(use_tc_tiling_on_sc=False),
    scratch_types=[
        pltpu.VMEM((_K3,), jnp.int32),
        pltpu.VMEM((_K3,), jnp.int32),
        pltpu.VMEM((_K3, 16), jnp.float32),
        pltpu.VMEM((_K3, 16), jnp.float32),
        pltpu.VMEM((_K3,), jnp.float32),
        pltpu.VMEM_SHARED((_NPAD, 16), jnp.float32),
        pltpu.VMEM_SHARED((_NPAD,), jnp.float32),
        pltpu.SemaphoreType.DMA,
        pltpu.SemaphoreType.DMA,
        pltpu.SemaphoreType.DMA,
    ],
)
def _gat1_kernel(asrc_hbm, adst_hbm, m_hbm, row_hbm, col_hbm, z16_hbm, zv_hbm,
                 ea_hbm, den_hbm, outm_hbm,
                 idx_r, idx_c, ar, bc, mg, accd, accm, sem, sem2, sem3):
    c = lax.axis_index("c")
    s = lax.axis_index("s")
    r0 = s * _RPT
    pltpu.sync_copy(z16_hbm.at[pl.ds(r0, _RPT)], accd.at[pl.ds(r0, _RPT)])
    pltpu.sync_copy(zv_hbm.at[pl.ds(r0, _RPT)], accm.at[pl.ds(r0, _RPT)])
    plsc.subcore_barrier()
    base0 = c * _EPC + s * _EPT

    def body(i, carry):
        base = base0 + i * _K3
        pltpu.sync_copy(row_hbm.at[pl.ds(base, _K3)], idx_r)
        pltpu.sync_copy(col_hbm.at[pl.ds(base, _K3)], idx_c)
        cp1 = pltpu.async_copy(asrc_hbm.at[idx_r], ar, sem)
        cp2 = pltpu.async_copy(adst_hbm.at[idx_c], bc, sem2)
        cp3 = pltpu.async_copy(m_hbm.at[idx_r], mg, sem3)
        cp1.wait()
        cp2.wait()
        cp3.wait()

        def vb(e, carry2):
            z = ar[e, :] + bc[e, :]
            z = jnp.where(z > 0.0, z, z * 0.2)
            ar[e, :] = jnp.exp(z)
            return carry2

        lax.fori_loop(0, _K3, vb, 0)
        pltpu.sync_copy(ar, accd.at[idx_c], add=True)
        pltpu.sync_copy(mg, accm.at[idx_c], add=True)
        pltpu.sync_copy(ar, ea_hbm.at[pl.ds(base, _K3)])
        return carry

    lax.fori_loop(0, _NCH3, body, 0)
    plsc.subcore_barrier()
    pltpu.sync_copy(accd.at[pl.ds(r0, _RPT)], den_hbm.at[c, pl.ds(r0, _RPT)])
    pltpu.sync_copy(accm.at[pl.ds(r0, _RPT)], outm_hbm.at[c, pl.ds(r0, _RPT)])


_K4 = 200
_NCH4 = _EPT // _K4


@functools.partial(
    pl.kernel,
    out_type=jax.ShapeDtypeStruct((4, _NC, _NPAD, H1), jnp.float32),
    mesh=_MESH,
    scratch_types=[
        pltpu.VMEM((_K4,), jnp.int32),
        pltpu.VMEM((_K4,), jnp.int32),
        pltpu.VMEM((_K4, H1), jnp.float32),
        pltpu.VMEM((_K4 * 16,), jnp.float32),
        pltpu.VMEM_SHARED((_NPAD, H1), jnp.float32),
        pltpu.SemaphoreType.DMA,
    ],
)
def _gat2_kernel(hh0_hbm, hh1_hbm, hh2_hbm, hh3_hbm, eaf_hbm, row_hbm, col_hbm,
                 zm_hbm, out_hbm, idx_r, idx_c, rows2, eaf, acc, sem):
    c = lax.axis_index("c")
    s = lax.axis_index("s")
    r0 = s * _RPT
    base0 = c * _EPC + s * _EPT
    tables = (hh0_hbm, hh1_hbm, hh2_hbm, hh3_hbm)

    for cnk in range(4):
        hh_hbm = tables[cnk]
        pltpu.sync_copy(zm_hbm.at[pl.ds(r0, _RPT)], acc.at[pl.ds(r0, _RPT)])
        plsc.subcore_barrier()

        def body(i, carry):
            base = base0 + i * _K4
            pltpu.sync_copy(row_hbm.at[pl.ds(base, _K4)], idx_r)
            cp1 = pltpu.async_copy(hh_hbm.at[idx_r], rows2, sem)
            pltpu.sync_copy(eaf_hbm.at[pl.ds(base * 16, _K4 * 16)], eaf)
            pltpu.sync_copy(col_hbm.at[pl.ds(base, _K4)], idx_c)
            cp1.wait()

            def eb(e, carry2):
                ev = eaf[pl.ds(e * 16, 16)]
                for j in range(8):
                    hidx = jnp.full((16,), 2 * cnk + j // 4, jnp.int32)
                    m = ev.at[hidx].get(mode="promise_in_bounds")
                    rows2[e, pl.ds(j * 16, 16)] = (
                        rows2[e, pl.ds(j * 16, 16)] * m)
                return carry2

            lax.fori_loop(0, _K4, eb, 0)
            pltpu.sync_copy(rows2, acc.at[idx_c], add=True)
            return carry

        lax.fori_loop(0, _NCH4, body, 0)
        plsc.subcore_barrier()
        pltpu.sync_copy(acc.at[pl.ds(r0, _RPT)],
                        out_hbm.at[cnk, c, pl.ds(r0, _RPT)])


_K5 = 1000
_NCH5 = _EPT // _K5


@functools.partial(
    pl.kernel,
    out_type=jax.ShapeDtypeStruct((_NC, 16, 16), jnp.float32),
    mesh=_MESH,
    compiler_params=pltpu.CompilerParams(use_tc_tiling_on_sc=False),
    scratch_types=[
        pltpu.VMEM((_K5,), jnp.int32),
        pltpu.VMEM((_K5,), jnp.int32),
        pltpu.VMEM((_K5, 16), jnp.float32),
        pltpu.VMEM((_K5, 16), jnp.float32),
        pltpu.VMEM((_K5,), jnp.int32),
        pltpu.VMEM((16, 16), jnp.float32),
        pltpu.VMEM_SHARED((16, 16), jnp.float32),
        pltpu.SemaphoreType.DMA,
        pltpu.SemaphoreType.DMA,
        pltpu.SemaphoreType.DMA,
    ],
)
def _cut_kernel(p_hbm, b_hbm, row_hbm, col_hbm,
                out_hbm, idx_r, idx_c, pr, pc, bg, zv, acc, sem, sem2, sem3):
    c = lax.axis_index("c")
    s = lax.axis_index("s")
    zvec16 = jnp.zeros((16,), jnp.float32)
    for i in range(16):
        zv[i, :] = zvec16

    @pl.when(s == 0)
    def _():
        pltpu.sync_copy(zv, acc)

    plsc.subcore_barrier()
    base0 = c * _EPC + s * _EPT

    def body(i, carry):
        base = base0 + i * _K5
        pltpu.sync_copy(row_hbm.at[pl.ds(base, _K5)], idx_r)
        pltpu.sync_copy(col_hbm.at[pl.ds(base, _K5)], idx_c)
        cp1 = pltpu.async_copy(p_hbm.at[idx_r], pr, sem)
        cp2 = pltpu.async_copy(p_hbm.at[idx_c], pc, sem2)
        cp3 = pltpu.async_copy(b_hbm.at[idx_r], bg, sem3)
        cp1.wait()
        cp2.wait()
        cp3.wait()

        def vb(e, carry2):
            pr[e, :] = pr[e, :] * pc[e, :]
            return carry2

        lax.fori_loop(0, _K5, vb, 0)
        pltpu.sync_copy(pr, acc.at[bg], add=True)
        return carry

    lax.fori_loop(0, _NCH5, body, 0)
    plsc.subcore_barrier()

    @pl.when(s == 0)
    def _():
        pltpu.sync_copy(acc, out_hbm.at[c])


def _niter_body(h_ref, mask_ref, xinit_ref, deg_ref, oh_ref, oht_ref,
                tvol_ref, probs_ref, cut1_ref):
    hv = h_ref[...]
    mv = mask_ref[...]
    xv = xinit_ref[...]
    dv = deg_ref[...]
    oh = oh_ref[...]
    oht = oht_ref[...]
    neg = jnp.float32(-jnp.inf)
    bmaxn = jnp.zeros_like(hv)
    bminn = jnp.zeros_like(hv)
    for g in range(G):
        sel = oht[g:g + 1, :]
        mg = jnp.max(jnp.where(sel > 0.0, hv, neg))
        mg = jnp.where(jnp.isfinite(mg), mg, 0.0)
        ng = -jnp.max(jnp.where(sel > 0.0, -hv, neg))
        ng = jnp.where(jnp.isfinite(ng), ng, 0.0)
        bmaxn = bmaxn + mg * sel
        bminn = bminn + ng * sel
    hv = (hv - bminn) / (bmaxn + 1e-06 - bminn)
    hv = hv * mv + mv * 1e-06 + xv
    totalvol = jnp.dot(dv, oh, preferred_element_type=jnp.float32) + 1e-06
    target = tvol_ref[...] * totalvol

    def it(i, a):
        an = jnp.dot(a, oht, preferred_element_type=jnp.float32)
        keep = (an * hv < 1.0).astype(jnp.float32)
        km = keep * mv
        xk = hv * km
        dk = dv * km
        dnk = dv * (1.0 - keep) * mv
        diff = target - jnp.dot(dnk, oh, preferred_element_type=jnp.float32)
        dot = jnp.dot(xk * dk, oh, preferred_element_type=jnp.float32)
        return diff / (dot + 1e-05)

    a = lax.fori_loop(0, NITER, it, jnp.ones((1, G), jnp.float32))
    an = jnp.dot(a, oht, preferred_element_type=jnp.float32)
    probs = jnp.clip(an * hv * mv, 0.0, 1.0)
    probs_ref[...] = probs
    cut1_ref[...] = jnp.dot(probs * dv, oh, preferred_element_type=jnp.float32)


def _niter_tc(h2, maskf, xinitf, degf, onehot, onehotT, tvol2):
    return pl.pallas_call(
        _niter_body,
        out_shape=(
            jax.ShapeDtypeStruct((1, _NPAD), jnp.float32),
            jax.ShapeDtypeStruct((1, G), jnp.float32),
        ),
    )(h2, maskf, xinitf, degf, onehot, onehotT, tvol2)


def kernel(x, edge_index, batch, tvol, p):
    row, col = edge_index[0], edge_index[1]
    xinit = x

    zvec = jnp.zeros((_NPAD,), jnp.float32)
    zmat = jnp.zeros((_NPAD, H1), jnp.float32)
    z16 = jnp.zeros((_NPAD, 16), jnp.float32)

    def padv(v):
        return jnp.zeros((_NPAD,), jnp.float32).at[:N].set(v)

    def padm(m):
        return jnp.zeros((_NPAD, H1), jnp.float32).at[:N].set(m)

    xflat = padv(x[:, 0])
    xaggp, mmp, degp = _edge_init_kernel(xflat, row, col, zvec)
    mm0 = jnp.abs(x[:, 0]) > 0
    neigh = mmp[0, :N] + mmp[1, :N]
    mask = jnp.logical_or(mm0, neigh > 0).astype(jnp.float32)[:, None]
    deg = (degp[0, :N] + degp[1, :N])[:, None]

    def mlp(agg, W1, b1, W2, b2, g, bt):
        z = jax.nn.relu(agg @ W1 + b1)
        z = jax.nn.relu(z @ W2 + b2)
        return z * g + bt

    xagg = x + (xaggp[0, :N] + xaggp[1, :N])[:, None]
    h = mlp(xagg, p['c1_W1'], p['c1_b1'], p['c1_W2'], p['c1_b2'], p['c1_g'], p['c1_bt'])
    h = h * mask

    for i in range(3):
        aggp, mmp = _gin_mask_kernel(padm(h), padv(mask[:, 0]), row, col, zmat, zvec)
        agg = h + aggp[0, :N] + aggp[1, :N]
        gi = mlp(agg, p['cv%d_W1' % i], p['cv%d_b1' % i], p['cv%d_W2' % i],
                 p['cv%d_b2' % i], p['cv%d_g' % i], p['cv%d_bt' % i])
        h = h + gi
        neigh = mmp[0, :N] + mmp[1, :N]
        mask = jnp.logical_or(mask[:, 0] > 0, neigh > 0).astype(jnp.float32)[:, None]
        h = h * mask
        h = h * p['bn%d_g' % i] + p['bn%d_b' % i]

    hh = (h @ p['gat_W']).reshape(N, HEADS, H2)
    asrc = (hh * p['gat_asrc']).sum(-1)
    adst = (hh * p['gat_adst']).sum(-1)
    a16 = jnp.zeros((_NPAD, 16), jnp.float32).at[:N].set(jnp.tile(asrc, (1, 2)))
    b16 = jnp.zeros((_NPAD, 16), jnp.float32).at[:N].set(jnp.tile(adst, (1, 2)))
    eaf, denp, mmp = _gat1_kernel(a16, b16, padv(mask[:, 0]), row, col, z16, zvec)
    den = denp[0, :N, :HEADS] + denp[1, :N, :HEADS] + 1e-16
    neigh = mmp[0, :N] + mmp[1, :N]
    mask = jnp.logical_or(mask[:, 0] > 0, neigh > 0).astype(jnp.float32)[:, None]

    eaflat = eaf.reshape(E * 16)
    hhcs = [jnp.zeros((_NPAD, H1), jnp.float32).at[:N].set(
        hh[:, 2 * cnk:2 * cnk + 2, :].reshape(N, H1)) for cnk in range(4)]
    nump = _gat2_kernel(hhcs[0], hhcs[1], hhcs[2], hhcs[3], eaflat,
                        row, col, zmat)
    num = [nump[cnk, 0, :N] + nump[cnk, 1, :N] for cnk in range(4)]
    numf = jnp.concatenate(num, axis=1).reshape(N, HEADS, H2)
    h = (numf / den[:, :, None]).reshape(N, HEADS * H2)
    h = h * mask
    h = jax.nn.leaky_relu(h @ p['l1_W'] + p['l1_b'])
    h = h * mask
    h = h * p['bn2_g'] + p['bn2_b']
    h = jax.nn.leaky_relu(h @ p['l2_W'] + p['l2_b'])
    h = h * mask

    onehot = (batch[:, None] == jnp.arange(G)[None, :]).astype(jnp.float32)
    onehot = jnp.zeros((_NPAD, G), jnp.float32).at[:N].set(onehot)
    onehotT = onehot.T
    probs2, cut1 = _niter_tc(
        padv(h[:, 0]).reshape(1, _NPAD),
        padv(mask[:, 0]).reshape(1, _NPAD),
        padv(xinit[:, 0]).reshape(1, _NPAD),
        padv(deg[:, 0]).reshape(1, _NPAD),
        onehot, onehotT, tvol.reshape(1, G))
    probs = probs2[0, :N]

    bpad = jnp.zeros((_NPAD,), jnp.int32).at[:N].set(batch)
    p16 = jnp.tile(probs2.reshape(_NPAD, 1), (1, 16))
    cutp = _cut_kernel(p16, bpad, row, col)
    cut2 = cutp[0, :, 0] + cutp[1, :, 0]
    expected_cut = cut1[0][:, None] - cut2[:, None]
    return probs, expected_cut
